# Initial kernel scaffold; baseline (speedup 1.0000x reference)
import jax
import jax.numpy as jnp
from jax.experimental import pallas as pl


def kernel(hidden_states, gate_weight, W_gate, W_up, W_down):
    raise NotImplementedError("write your pallas kernel here")


# speedup vs baseline: 1.4768x; 1.4768x over previous
import functools

import jax
import jax.numpy as jnp
from jax.experimental import pallas as pl
from jax.experimental.pallas import tpu as pltpu

HIDDEN = 1024
INTER = 768
NUM_EXPERTS = 8
TOP_K = 2


def _router_body(x_ref, gw_ref, w_ref):
    xb = x_ref[...]
    logits = jax.lax.dot_general(
        xb, gw_ref[...], (((1,), (1,)), ((), ())),
        preferred_element_type=jnp.float32,
    )
    m = jnp.max(logits, axis=-1, keepdims=True)
    ex = jnp.exp(logits - m)
    probs = ex / jnp.sum(ex, axis=-1, keepdims=True)
    ii = jax.lax.broadcasted_iota(jnp.int32, probs.shape, 1)
    m1 = jnp.max(probs, axis=-1, keepdims=True)
    i1 = jnp.min(jnp.where(probs == m1, ii, NUM_EXPERTS), axis=-1, keepdims=True)
    sel1 = ii == i1
    probs2 = jnp.where(sel1, -jnp.inf, probs)
    m2 = jnp.max(probs2, axis=-1, keepdims=True)
    i2 = jnp.min(jnp.where(probs2 == m2, ii, NUM_EXPERTS), axis=-1, keepdims=True)
    sel2 = ii == i2
    denom = m1 + m2
    w_ref[...] = jnp.where(sel1, m1 / denom, jnp.where(sel2, m2 / denom, 0.0))


def _moe_body(w_ref, x_ref, wg_ref, wu_ref, wd_ref, o_ref):
    e = pl.program_id(0)
    xb = x_ref[...]
    g = jnp.dot(xb, wg_ref[0], preferred_element_type=jnp.float32)
    u = jnp.dot(xb, wu_ref[0], preferred_element_type=jnp.float32)
    a = (g * jax.nn.sigmoid(g) * u).astype(jnp.bfloat16)
    y = jnp.dot(a, wd_ref[0], preferred_element_type=jnp.float32)
    wf = w_ref[...]
    ii = jax.lax.broadcasted_iota(jnp.int32, wf.shape, 1)
    w = jnp.sum(jnp.where(ii == e, wf, 0.0), axis=-1, keepdims=True)
    contrib = w * y

    @pl.when(e == 0)
    def _init():
        o_ref[...] = contrib

    @pl.when(e > 0)
    def _acc():
        o_ref[...] += contrib


def kernel(hidden_states, gate_weight, W_gate, W_up, W_down):
    b, s, h = hidden_states.shape
    x = hidden_states.reshape(-1, h)
    T = x.shape[0]

    w_full = pl.pallas_call(
        _router_body,
        grid=(T // 256,),
        in_specs=[
            pl.BlockSpec((256, HIDDEN), lambda t: (t, 0)),
            pl.BlockSpec((NUM_EXPERTS, HIDDEN), lambda t: (0, 0)),
        ],
        out_specs=pl.BlockSpec((256, NUM_EXPERTS), lambda t: (t, 0)),
        out_shape=jax.ShapeDtypeStruct((T, NUM_EXPERTS), jnp.float32),
    )(x, gate_weight)

    x16 = x.astype(jnp.bfloat16)
    wg16 = W_gate.astype(jnp.bfloat16)
    wu16 = W_up.astype(jnp.bfloat16)
    wd16 = W_down.astype(jnp.bfloat16)

    out = pl.pallas_call(
        _moe_body,
        grid=(NUM_EXPERTS,),
        in_specs=[
            pl.BlockSpec((T, NUM_EXPERTS), lambda e: (0, 0)),
            pl.BlockSpec((T, HIDDEN), lambda e: (0, 0)),
            pl.BlockSpec((1, HIDDEN, INTER), lambda e: (e, 0, 0)),
            pl.BlockSpec((1, HIDDEN, INTER), lambda e: (e, 0, 0)),
            pl.BlockSpec((1, INTER, HIDDEN), lambda e: (e, 0, 0)),
        ],
        out_specs=pl.BlockSpec((T, HIDDEN), lambda e: (0, 0)),
        out_shape=jax.ShapeDtypeStruct((T, HIDDEN), jnp.float32),
        compiler_params=pltpu.CompilerParams(
            dimension_semantics=("arbitrary",),
        ),
    )(w_full, x16, wg16, wu16, wd16)

    return out.reshape(b, s, h)
